# baseline (speedup 1.0000x reference)
import functools

import jax
import jax.numpy as jnp
import numpy as np
from jax import lax
from jax.experimental import pallas as pl
from jax.experimental.pallas import tpu as pltpu
from jax.experimental.pallas import tpu_sc as plsc

K = 32
ITERS = 10
L = 16


def _kmeans_body(x_hbm, c0_hbm, out_hbm, xv, consts, acc2, acct, red, cents,
                 g4, out96, shared):
    c = lax.axis_index("c")
    s = lax.axis_index("s")
    img = c * 4 + s // 4
    part = s % 4
    NP = xv.shape[0] // 3
    N = NP * 4
    n_groups = NP // L

    for j in range(3):
        pltpu.sync_copy(
            x_hbm.at[pl.ds(img * 3 * N + j * N + part * NP, NP)],
            xv.at[pl.ds(j * NP, NP)])
    pltpu.sync_copy(c0_hbm.at[pl.ds(img * 96, 96)], cents.at[pl.ds(0, 96)])

    lane = lax.iota(jnp.int32, L)
    ones = jnp.full((L,), 1.0, jnp.float32)
    big = jnp.full((L,), 3.4e38, jnp.float32)

    for h in range(2):
        c0 = cents[pl.ds(h * L, L)]
        c1 = cents[pl.ds(32 + h * L, L)]
        c2 = cents[pl.ds(64 + h * L, L)]
        cents[pl.ds(96 + h * L, L)] = c0 * c0 + c1 * c1 + c2 * c2

    def iter_body(it, carry):
        for h in range(2):
            av = cents[pl.ds(96 + h * L, L)]
            bv = [cents[pl.ds(j * 32 + h * L, L)] * (-2.0) for j in range(3)]
            for kk in range(L):
                k = h * L + kk
                consts[pl.ds(k * 64, L)] = jnp.broadcast_to(av[kk], (L,))
                for j in range(3):
                    consts[pl.ds(k * 64 + (1 + j) * L, L)] = (
                        jnp.broadcast_to(bv[j][kk], (L,)))

        zero = jnp.zeros((L,), jnp.float32)
        for r in range(L):
            for cc in range(8):
                acc2[pl.ds(r * 128 + cc * L, L)] = zero

        UN = 4

        def gbody(g):
            xs = [[xv[pl.ds(j * NP + (UN * g + u) * L, L)] for j in range(3)]
                  for u in range(UN)]
            best = [big] * UN
            bidx = [jnp.zeros((L,), jnp.int32)] * UN
            for k in range(K):
                a = consts[pl.ds(k * 64, L)]
                b0 = consts[pl.ds(k * 64 + L, L)]
                b1 = consts[pl.ds(k * 64 + 2 * L, L)]
                b2 = consts[pl.ds(k * 64 + 3 * L, L)]
                kv = jnp.full((L,), k, jnp.int32)
                for u in range(UN):
                    s = xs[u][0] * b0 + (xs[u][1] * b1 + (xs[u][2] * b2 + a))
                    m = s < best[u]
                    best[u] = jnp.minimum(best[u], s)
                    bidx[u] = jnp.where(m, kv, bidx[u])
            for u in range(UN):
                fl = bidx[u] * L + lane
                plsc.addupdate_scatter(acc2, [fl], xs[u][0])
                plsc.addupdate_scatter(acc2, [fl + 512], xs[u][1])
                plsc.addupdate_scatter(acc2, [fl + 1024], xs[u][2])
                plsc.addupdate_scatter(acc2, [fl + 1536], ones)

        plsc.parallel_loop(0, n_groups // UN, 1, unroll=1)(gbody)

        for p in range(128):
            v = acc2[pl.ds(p * L, L)]
            plsc.store_scatter(acct, [lane * 128 + p], v)
        for cc in range(8):
            t = acct[pl.ds(cc * L, L)]
            for r in range(1, L):
                t = t + acct[pl.ds(r * 128 + cc * L, L)]
            red[pl.ds(cc * L, L)] = t

        pltpu.sync_copy(red, shared.at[pl.ds(s * 128, 128)])
        plsc.subcore_barrier()
        pltpu.sync_copy(shared.at[pl.ds((s // 4) * 4 * 128, 512)], g4)

        for h in range(2):
            sums = []
            for j in range(4):
                o = j * 32 + h * L
                sums.append(g4[pl.ds(o, L)] + g4[pl.ds(128 + o, L)]
                            + g4[pl.ds(256 + o, L)] + g4[pl.ds(384 + o, L)])
            cnt = sums[3]
            denom = jnp.maximum(cnt, 1.0)
            m = cnt > 0.0
            newc = []
            for j in range(3):
                o = j * 32 + h * L
                nc = jnp.where(m, sums[j] / denom, cents[pl.ds(o, L)])
                newc.append(nc)
                cents[pl.ds(o, L)] = nc
            cents[pl.ds(96 + h * L, L)] = (
                newc[0] * newc[0] + newc[1] * newc[1] + newc[2] * newc[2])
        plsc.subcore_barrier()
        return carry

    lax.fori_loop(0, ITERS, iter_body, 0)

    @pl.when(part == 0)
    def _():
        for j in range(3):
            for h in range(2):
                idx = (lane + h * L) * 3 + j
                plsc.store_scatter(out96, [idx],
                                   cents[pl.ds(j * 32 + h * L, L)])
        pltpu.sync_copy(out96, out_hbm.at[pl.ds(img * 96, 96)])


@jax.jit
def _sc_kmeans(x_flat, c_flat):
    B = 8
    N = x_flat.shape[0] // (3 * B)
    NP = N // 4
    mesh = plsc.VectorSubcoreMesh(core_axis_name="c", subcore_axis_name="s")
    f = functools.partial(
        pl.kernel,
        mesh=mesh,
        out_type=jax.ShapeDtypeStruct((B * K * 3,), jnp.float32),
        compiler_params=pltpu.CompilerParams(needs_layout_passes=False),
        scratch_types=[
            pltpu.VMEM((3 * NP,), jnp.float32),
            pltpu.VMEM((K * 4 * L,), jnp.float32),
            pltpu.VMEM((L * 8 * L,), jnp.float32),
            pltpu.VMEM((L * 8 * L,), jnp.float32),
            pltpu.VMEM((8 * L,), jnp.float32),
            pltpu.VMEM((8 * L,), jnp.float32),
            pltpu.VMEM((4 * 8 * L,), jnp.float32),
            pltpu.VMEM((K * 3,), jnp.float32),
            pltpu.VMEM_SHARED((L * 8 * L,), jnp.float32),
        ],
    )(_kmeans_body)
    return f(x_flat, c_flat)


_PERM_CACHE = {}


def _get_perms(b, n):
    if (b, n) not in _PERM_CACHE:
        with jax.ensure_compile_time_eval():
            sk = jax.random.split(jax.random.key(42), b)
            p = jax.vmap(lambda k: jax.random.permutation(k, n)[:K])(sk)
            _PERM_CACHE[(b, n)] = np.asarray(p)
    return _PERM_CACHE[(b, n)]


def kernel(inputs):
    B, H, W, C = inputs.shape
    N = H * W
    x = inputs.reshape(B, N, C)
    perms = jnp.asarray(_get_perms(B, N))
    cents0 = jnp.take_along_axis(x, perms[:, :, None], axis=1)
    x_flat = x.transpose(0, 2, 1).reshape(-1)
    c_flat = cents0.transpose(0, 2, 1).reshape(-1)
    out = _sc_kmeans(x_flat, c_flat)
    return out.reshape(B, K * 3)


# speedup vs baseline: 1.3260x; 1.3260x over previous
import functools

import jax
import jax.numpy as jnp
import numpy as np
from jax import lax
from jax.experimental import pallas as pl
from jax.experimental.pallas import tpu as pltpu
from jax.experimental.pallas import tpu_sc as plsc

K = 32
ITERS = 10
L = 16


def _kmeans_body(x_hbm, c0_hbm, out_hbm, xv, consts, acc2, red, cents, g4,
                 out96, shared):
    c = lax.axis_index("c")
    s = lax.axis_index("s")
    img = c * 4 + s // 4
    part = s % 4
    NP = xv.shape[0] // 3
    N = NP * 4
    n_groups = NP // L

    for j in range(3):
        pltpu.sync_copy(
            x_hbm.at[pl.ds(img * 3 * N + j * N + part * NP, NP)],
            xv.at[pl.ds(j * NP, NP)])
    pltpu.sync_copy(c0_hbm.at[pl.ds(img * 96, 96)], cents.at[pl.ds(0, 96)])

    lane = lax.iota(jnp.int32, L)
    ones = jnp.full((L,), 1.0, jnp.float32)
    big = jnp.full((L,), 3.4e38, jnp.float32)

    for h in range(2):
        c0 = cents[pl.ds(h * L, L)]
        c1 = cents[pl.ds(32 + h * L, L)]
        c2 = cents[pl.ds(64 + h * L, L)]
        cents[pl.ds(96 + h * L, L)] = c0 * c0 + c1 * c1 + c2 * c2

    def iter_body(it, carry):
        for h in range(2):
            av = cents[pl.ds(96 + h * L, L)]
            bv = [cents[pl.ds(j * 32 + h * L, L)] * (-2.0) for j in range(3)]
            for kk in range(L):
                k = h * L + kk
                consts[pl.ds(k * 64, L)] = jnp.broadcast_to(av[kk], (L,))
                for j in range(3):
                    consts[pl.ds(k * 64 + (1 + j) * L, L)] = (
                        jnp.broadcast_to(bv[j][kk], (L,)))

        zero = jnp.zeros((L,), jnp.float32)
        for r in range(L):
            for cc in range(8):
                acc2[pl.ds(r * 128 + cc * L, L)] = zero

        UN = 4

        def gbody(g):
            xs = [[xv[pl.ds(j * NP + (UN * g + u) * L, L)] for j in range(3)]
                  for u in range(UN)]
            best = [big] * UN
            bidx = [jnp.zeros((L,), jnp.int32)] * UN
            for k in range(K):
                a = consts[pl.ds(k * 64, L)]
                b0 = consts[pl.ds(k * 64 + L, L)]
                b1 = consts[pl.ds(k * 64 + 2 * L, L)]
                b2 = consts[pl.ds(k * 64 + 3 * L, L)]
                kv = jnp.full((L,), k, jnp.int32)
                for u in range(UN):
                    s = xs[u][0] * b0 + (xs[u][1] * b1 + (xs[u][2] * b2 + a))
                    m = s < best[u]
                    best[u] = jnp.where(m, s, best[u])
                    bidx[u] = jnp.where(m, kv, bidx[u])
            for u in range(UN):
                fl = lane * 128 + bidx[u]
                plsc.addupdate_scatter(acc2, [fl], xs[u][0])
                plsc.addupdate_scatter(acc2, [fl + 32], xs[u][1])
                plsc.addupdate_scatter(acc2, [fl + 64], xs[u][2])
                plsc.addupdate_scatter(acc2, [fl + 96], ones)

        plsc.parallel_loop(0, n_groups // UN, 1, unroll=1)(gbody)

        for cc in range(8):
            t = acc2[pl.ds(cc * L, L)]
            for r in range(1, L):
                t = t + acc2[pl.ds(r * 128 + cc * L, L)]
            red[pl.ds(cc * L, L)] = t

        pltpu.sync_copy(red, shared.at[pl.ds(s * 128, 128)])
        plsc.subcore_barrier()
        pltpu.sync_copy(shared.at[pl.ds((s // 4) * 4 * 128, 512)], g4)

        for h in range(2):
            sums = []
            for j in range(4):
                o = j * 32 + h * L
                sums.append(g4[pl.ds(o, L)] + g4[pl.ds(128 + o, L)]
                            + g4[pl.ds(256 + o, L)] + g4[pl.ds(384 + o, L)])
            cnt = sums[3]
            denom = jnp.maximum(cnt, 1.0)
            m = cnt > 0.0
            newc = []
            for j in range(3):
                o = j * 32 + h * L
                nc = jnp.where(m, sums[j] / denom, cents[pl.ds(o, L)])
                newc.append(nc)
                cents[pl.ds(o, L)] = nc
            cents[pl.ds(96 + h * L, L)] = (
                newc[0] * newc[0] + newc[1] * newc[1] + newc[2] * newc[2])
        plsc.subcore_barrier()
        return carry

    lax.fori_loop(0, ITERS, iter_body, 0)

    @pl.when(part == 0)
    def _():
        for j in range(3):
            for h in range(2):
                idx = (lane + h * L) * 3 + j
                plsc.store_scatter(out96, [idx],
                                   cents[pl.ds(j * 32 + h * L, L)])
        pltpu.sync_copy(out96, out_hbm.at[pl.ds(img * 96, 96)])


@jax.jit
def _sc_kmeans(x_flat, c_flat):
    B = 8
    N = x_flat.shape[0] // (3 * B)
    NP = N // 4
    mesh = plsc.VectorSubcoreMesh(core_axis_name="c", subcore_axis_name="s")
    f = functools.partial(
        pl.kernel,
        mesh=mesh,
        out_type=jax.ShapeDtypeStruct((B * K * 3,), jnp.float32),
        compiler_params=pltpu.CompilerParams(needs_layout_passes=False),
        scratch_types=[
            pltpu.VMEM((3 * NP,), jnp.float32),
            pltpu.VMEM((K * 4 * L,), jnp.float32),
            pltpu.VMEM((L * 8 * L,), jnp.float32),
            pltpu.VMEM((8 * L,), jnp.float32),
            pltpu.VMEM((8 * L,), jnp.float32),
            pltpu.VMEM((4 * 8 * L,), jnp.float32),
            pltpu.VMEM((K * 3,), jnp.float32),
            pltpu.VMEM_SHARED((L * 8 * L,), jnp.float32),
        ],
    )(_kmeans_body)
    return f(x_flat, c_flat)


_PERM_CACHE = {}


def _get_perms(b, n):
    if (b, n) not in _PERM_CACHE:
        with jax.ensure_compile_time_eval():
            sk = jax.random.split(jax.random.key(42), b)
            p = jax.vmap(lambda k: jax.random.permutation(k, n)[:K])(sk)
            _PERM_CACHE[(b, n)] = np.asarray(p)
    return _PERM_CACHE[(b, n)]


def kernel(inputs):
    B, H, W, C = inputs.shape
    N = H * W
    x = inputs.reshape(B, N, C)
    perms = jnp.asarray(_get_perms(B, N))
    cents0 = jnp.take_along_axis(x, perms[:, :, None], axis=1)
    x_flat = x.transpose(0, 2, 1).reshape(-1)
    c_flat = cents0.transpose(0, 2, 1).reshape(-1)
    out = _sc_kmeans(x_flat, c_flat)
    return out.reshape(B, K * 3)
